# baseline (speedup 1.0000x reference)
import jax
import jax.numpy as jnp
from jax import lax
from jax.experimental import pallas as pl
from jax.experimental.pallas import tpu as pltpu
from jax.experimental.pallas import tpu_sc as plsc

N_NODES = 10000
N_EDGES = 320000
D = 128

NC = 2
NS = 16
NW = NC * NS
LANE = 64
GR = 16
CH = 160
E_PAD = NW * CH * LANE
CH_A = 320
CH_B = 2 * CH - CH_A
N_PAD = 10240
STRIPE = N_PAD // NS
K = STRIPE // LANE


def _tc_mm_body(x_ref, wl_ref, wr_ref, b_ref, y_ref, z_ref):
    xb = x_ref[...]
    y_ref[...] = jnp.dot(xb, wl_ref[...], preferred_element_type=jnp.float32)
    z_ref[...] = (
        jnp.dot(xb, wr_ref[...], preferred_element_type=jnp.float32) + b_ref[...]
    )


def _tc_mm(xp, Wl, Wr, b):
    blk = 1024
    return pl.pallas_call(
        _tc_mm_body,
        grid=(N_PAD // blk,),
        in_specs=[
            pl.BlockSpec((blk, D), lambda i: (i, 0)),
            pl.BlockSpec((D, D), lambda i: (0, 0)),
            pl.BlockSpec((D, D), lambda i: (0, 0)),
            pl.BlockSpec((1, D), lambda i: (0, 0)),
        ],
        out_specs=[
            pl.BlockSpec((blk, D), lambda i: (i, 0)),
            pl.BlockSpec((blk, D), lambda i: (i, 0)),
        ],
        out_shape=[
            jax.ShapeDtypeStruct((N_PAD, D), jnp.float32),
            jax.ShapeDtypeStruct((N_PAD, D), jnp.float32),
        ],
    )(xp, Wl, Wr, b)


def _tc_comb_mm_body(a_ref, d_ref, z_ref, wl_ref, wr_ref, b_ref, y_ref, z2_ref):
    deg = d_ref[0, :, 0:1] + d_ref[1, :, 0:1]
    agg = (a_ref[0] + a_ref[1]) / jnp.maximum(deg, 1.0)
    h = jnp.maximum(agg + z_ref[...], 0.0)
    y_ref[...] = jnp.dot(h, wl_ref[...], preferred_element_type=jnp.float32)
    z2_ref[...] = (
        jnp.dot(h, wr_ref[...], preferred_element_type=jnp.float32) + b_ref[...]
    )


def _tc_comb_mm(acc, dega, z, Wl, Wr, b):
    blk = 1024
    return pl.pallas_call(
        _tc_comb_mm_body,
        grid=(N_PAD // blk,),
        in_specs=[
            pl.BlockSpec((2, blk, D), lambda i: (0, i, 0)),
            pl.BlockSpec((2, blk, D), lambda i: (0, i, 0)),
            pl.BlockSpec((blk, D), lambda i: (i, 0)),
            pl.BlockSpec((D, D), lambda i: (0, 0)),
            pl.BlockSpec((D, D), lambda i: (0, 0)),
            pl.BlockSpec((1, D), lambda i: (0, 0)),
        ],
        out_specs=[
            pl.BlockSpec((blk, D), lambda i: (i, 0)),
            pl.BlockSpec((blk, D), lambda i: (i, 0)),
        ],
        out_shape=[
            jax.ShapeDtypeStruct((N_PAD, D), jnp.float32),
            jax.ShapeDtypeStruct((N_PAD, D), jnp.float32),
        ],
    )(acc, dega, z, Wl, Wr, b)


def _tc_final_body(a_ref, d_ref, z_ref, h_ref):
    deg = d_ref[0, :, 0:1] + d_ref[1, :, 0:1]
    agg = (a_ref[0] + a_ref[1]) / jnp.maximum(deg, 1.0)
    h_ref[...] = jnp.maximum(agg + z_ref[...], 0.0)


def _tc_final(acc, dega, z):
    blk = 1024
    return pl.pallas_call(
        _tc_final_body,
        grid=(N_PAD // blk,),
        in_specs=[
            pl.BlockSpec((2, blk, D), lambda i: (0, i, 0)),
            pl.BlockSpec((2, blk, D), lambda i: (0, i, 0)),
            pl.BlockSpec((blk, D), lambda i: (i, 0)),
        ],
        out_specs=pl.BlockSpec((blk, D), lambda i: (i, 0)),
        out_shape=jax.ShapeDtypeStruct((N_PAD, D), jnp.float32),
    )(acc, dega, z)


_MESH = plsc.VectorSubcoreMesh(core_axis_name="c", subcore_axis_name="s")
_SC_OUT = jax.ShapeDtypeStruct((NC, N_PAD, D), jnp.float32)


NB = 4


def _sc_agg_body(y_hbm, src_hbm, dst_hbm, zeros_hbm, acc_out,
                 acc_s, src_v, dst_v, *bufsems):
    bufs = bufsems[:NB]
    sems = bufsems[NB:]
    cid = lax.axis_index("c")
    sid = lax.axis_index("s")
    wid = cid * NS + sid
    r0 = sid * STRIPE

    pltpu.sync_copy(zeros_hbm, bufs[0])
    for k in range(K):
        pltpu.sync_copy(bufs[0], acc_s.at[pl.ds(r0 + k * LANE, LANE)])
    plsc.subcore_barrier()

    def group(g, carry):
        pltpu.sync_copy(src_hbm.at[wid, pl.ds(g * GR, GR)], src_v)
        pltpu.sync_copy(dst_hbm.at[wid, pl.ds(g * GR, GR)], dst_v)
        for j in range(NB - 1):
            pltpu.async_copy(y_hbm.at[src_v.at[j]], bufs[j], sems[j])
        for j in range(GR):
            cur, csem = bufs[j % NB], sems[j % NB]
            if j + NB - 1 < GR:
                pltpu.async_copy(
                    y_hbm.at[src_v.at[j + NB - 1]], bufs[(j + NB - 1) % NB],
                    sems[(j + NB - 1) % NB])
            pltpu.make_async_copy(y_hbm.at[src_v.at[j]], cur, csem).wait()
            pltpu.sync_copy(cur, acc_s.at[dst_v.at[j]], add=True)
        return carry

    ngroups = jnp.where(cid == 0, CH_A // GR, CH_B // GR)
    lax.fori_loop(0, ngroups, group, 0)
    plsc.subcore_barrier()

    for k in range(K):
        pltpu.sync_copy(acc_s.at[pl.ds(r0 + k * LANE, LANE)], bufs[0])
        pltpu.sync_copy(bufs[0], acc_out.at[cid, pl.ds(r0 + k * LANE, LANE)])


_sc_agg = pl.kernel(
    _sc_agg_body,
    out_type=_SC_OUT,
    mesh=_MESH,
    scratch_types=[
        pltpu.VMEM_SHARED((N_PAD, D), jnp.float32),
        pltpu.VMEM((GR, LANE), jnp.int32),
        pltpu.VMEM((GR, LANE), jnp.int32),
    ]
    + [pltpu.VMEM((LANE, D), jnp.float32) for _ in range(NB)]
    + [pltpu.SemaphoreType.DMA for _ in range(NB)],
)


def _sc_deg_body(dst_hbm, zeros_hbm, ones_hbm, acc_out,
                 acc_s, dst_v, rows_v):
    cid = lax.axis_index("c")
    sid = lax.axis_index("s")
    wid = cid * NS + sid
    r0 = sid * STRIPE

    pltpu.sync_copy(zeros_hbm, rows_v)
    for k in range(K):
        pltpu.sync_copy(rows_v, acc_s.at[pl.ds(r0 + k * LANE, LANE)])
    plsc.subcore_barrier()

    pltpu.sync_copy(ones_hbm, rows_v)

    def group(g, carry):
        pltpu.sync_copy(dst_hbm.at[wid, pl.ds(g * GR, GR)], dst_v)
        for j in range(GR):
            pltpu.sync_copy(rows_v, acc_s.at[dst_v.at[j]], add=True)
        return carry

    lax.fori_loop(0, CH // GR, group, 0)
    plsc.subcore_barrier()

    for k in range(K):
        pltpu.sync_copy(acc_s.at[pl.ds(r0 + k * LANE, LANE)], rows_v)
        pltpu.sync_copy(rows_v, acc_out.at[cid, pl.ds(r0 + k * LANE, LANE)])


_sc_deg = pl.kernel(
    _sc_deg_body,
    out_type=_SC_OUT,
    mesh=_MESH,
    scratch_types=[
        pltpu.VMEM_SHARED((N_PAD, D), jnp.float32),
        pltpu.VMEM((GR, LANE), jnp.int32),
        pltpu.VMEM((LANE, D), jnp.float32),
    ],
)


def kernel(x, edge_index, W1l, b1l, W1r, W2l, b2l, W2r):
    src = edge_index[0].astype(jnp.int32)
    dst = edge_index[1].astype(jnp.int32)
    pad = E_PAD - N_EDGES
    src_f = jnp.concatenate([src, jnp.zeros((pad,), jnp.int32)])
    dst_f = jnp.concatenate([dst, jnp.full((pad,), N_NODES, jnp.int32)])
    dst_p = dst_f.reshape(NW, CH, LANE)

    CHMX = max(CH_A, CH_B)

    def _split(arr, padval):
        cut = NS * CH_A * LANE
        a = arr[:cut].reshape(NS, CH_A, LANE)
        b = arr[cut:].reshape(NS, CH_B, LANE)
        if CH_A < CHMX:
            a = jnp.concatenate(
                [a, jnp.full((NS, CHMX - CH_A, LANE), padval, jnp.int32)], 1)
        if CH_B < CHMX:
            b = jnp.concatenate(
                [b, jnp.full((NS, CHMX - CH_B, LANE), padval, jnp.int32)], 1)
        return jnp.concatenate([a, b], axis=0)

    src_p = _split(src_f, 0)
    dst_p2 = _split(dst_f, N_NODES)

    xp = jnp.pad(x, ((0, N_PAD - N_NODES), (0, 0)))
    zeros = jnp.zeros((LANE, D), jnp.float32)
    ones = jnp.ones((LANE, D), jnp.float32)
    b1 = b1l.reshape(1, D)
    b2 = b2l.reshape(1, D)

    dega = _sc_deg(dst_p, zeros, ones)

    y1, z1 = _tc_mm(xp, W1l, W1r, b1)
    acc1 = _sc_agg(y1, src_p, dst_p2, zeros)
    y2, z2 = _tc_comb_mm(acc1, dega, z1, W2l, W2r, b2)
    acc2 = _sc_agg(y2, src_p, dst_p2, zeros)
    h2 = _tc_final(acc2, dega, z2)
    return h2[:N_NODES]


# speedup vs baseline: 1.2255x; 1.2255x over previous
import jax
import jax.numpy as jnp
from jax import lax
from jax.experimental import pallas as pl
from jax.experimental.pallas import tpu as pltpu
from jax.experimental.pallas import tpu_sc as plsc

N_NODES = 10000
N_EDGES = 320000
D = 128

NC = 2
NS = 16
NW = NC * NS
LANE = 64
GR = 16
CH = 160
E_PAD = NW * CH * LANE
CH_A = 256
CH_B = 2 * CH - CH_A
N_PAD = 10240
STRIPE = N_PAD // NS
K = STRIPE // LANE


def _tc_mm_body(x_ref, wl_ref, wr_ref, b_ref, y_ref, z_ref):
    xb = x_ref[...]
    y_ref[...] = jnp.dot(xb, wl_ref[...], preferred_element_type=jnp.float32)
    z_ref[...] = (
        jnp.dot(xb, wr_ref[...], preferred_element_type=jnp.float32) + b_ref[...]
    )


def _tc_mm(xp, Wl, Wr, b):
    blk = 1024
    return pl.pallas_call(
        _tc_mm_body,
        grid=(N_PAD // blk,),
        in_specs=[
            pl.BlockSpec((blk, D), lambda i: (i, 0)),
            pl.BlockSpec((D, D), lambda i: (0, 0)),
            pl.BlockSpec((D, D), lambda i: (0, 0)),
            pl.BlockSpec((1, D), lambda i: (0, 0)),
        ],
        out_specs=[
            pl.BlockSpec((blk, D), lambda i: (i, 0)),
            pl.BlockSpec((blk, D), lambda i: (i, 0)),
        ],
        out_shape=[
            jax.ShapeDtypeStruct((N_PAD, D), jnp.float32),
            jax.ShapeDtypeStruct((N_PAD, D), jnp.float32),
        ],
    )(xp, Wl, Wr, b)


def _tc_comb_mm_body(a_ref, d_ref, z_ref, wl_ref, wr_ref, b_ref, y_ref, z2_ref):
    deg = d_ref[0, :, 0:1] + d_ref[1, :, 0:1]
    agg = (a_ref[0] + a_ref[1]) / jnp.maximum(deg, 1.0)
    h = jnp.maximum(agg + z_ref[...], 0.0)
    y_ref[...] = jnp.dot(h, wl_ref[...], preferred_element_type=jnp.float32)
    z2_ref[...] = (
        jnp.dot(h, wr_ref[...], preferred_element_type=jnp.float32) + b_ref[...]
    )


def _tc_comb_mm(acc, dega, z, Wl, Wr, b):
    blk = 1024
    return pl.pallas_call(
        _tc_comb_mm_body,
        grid=(N_PAD // blk,),
        in_specs=[
            pl.BlockSpec((2, blk, D), lambda i: (0, i, 0)),
            pl.BlockSpec((2, blk, D), lambda i: (0, i, 0)),
            pl.BlockSpec((blk, D), lambda i: (i, 0)),
            pl.BlockSpec((D, D), lambda i: (0, 0)),
            pl.BlockSpec((D, D), lambda i: (0, 0)),
            pl.BlockSpec((1, D), lambda i: (0, 0)),
        ],
        out_specs=[
            pl.BlockSpec((blk, D), lambda i: (i, 0)),
            pl.BlockSpec((blk, D), lambda i: (i, 0)),
        ],
        out_shape=[
            jax.ShapeDtypeStruct((N_PAD, D), jnp.float32),
            jax.ShapeDtypeStruct((N_PAD, D), jnp.float32),
        ],
    )(acc, dega, z, Wl, Wr, b)


def _tc_final_body(a_ref, d_ref, z_ref, h_ref):
    deg = d_ref[0, :, 0:1] + d_ref[1, :, 0:1]
    agg = (a_ref[0] + a_ref[1]) / jnp.maximum(deg, 1.0)
    h_ref[...] = jnp.maximum(agg + z_ref[...], 0.0)


def _tc_final(acc, dega, z):
    blk = 1024
    return pl.pallas_call(
        _tc_final_body,
        grid=(N_PAD // blk,),
        in_specs=[
            pl.BlockSpec((2, blk, D), lambda i: (0, i, 0)),
            pl.BlockSpec((2, blk, D), lambda i: (0, i, 0)),
            pl.BlockSpec((blk, D), lambda i: (i, 0)),
        ],
        out_specs=pl.BlockSpec((blk, D), lambda i: (i, 0)),
        out_shape=jax.ShapeDtypeStruct((N_PAD, D), jnp.float32),
    )(acc, dega, z)


_MESH = plsc.VectorSubcoreMesh(core_axis_name="c", subcore_axis_name="s")
_SC_OUT = jax.ShapeDtypeStruct((NC, N_PAD, D), jnp.float32)


NB = 4


def _sc_agg_body(y_hbm, src_hbm, dst_hbm, zeros_hbm, acc_out,
                 acc_s, src_v, dst_v, *bufsems):
    bufs = bufsems[:NB]
    sems = bufsems[NB:]
    cid = lax.axis_index("c")
    sid = lax.axis_index("s")
    wid = cid * NS + sid
    r0 = sid * STRIPE

    pltpu.sync_copy(zeros_hbm, bufs[0])
    for k in range(K):
        pltpu.sync_copy(bufs[0], acc_s.at[pl.ds(r0 + k * LANE, LANE)])
    plsc.subcore_barrier()

    def group(g, carry):
        pltpu.sync_copy(src_hbm.at[wid, pl.ds(g * GR, GR)], src_v)
        pltpu.sync_copy(dst_hbm.at[wid, pl.ds(g * GR, GR)], dst_v)
        for j in range(NB - 1):
            pltpu.async_copy(y_hbm.at[src_v.at[j]], bufs[j], sems[j])
        for j in range(GR):
            cur, csem = bufs[j % NB], sems[j % NB]
            if j + NB - 1 < GR:
                pltpu.async_copy(
                    y_hbm.at[src_v.at[j + NB - 1]], bufs[(j + NB - 1) % NB],
                    sems[(j + NB - 1) % NB])
            pltpu.make_async_copy(y_hbm.at[src_v.at[j]], cur, csem).wait()
            pltpu.sync_copy(cur, acc_s.at[dst_v.at[j]], add=True)
        return carry

    ngroups = jnp.where(cid == 0, CH_A // GR, CH_B // GR)
    lax.fori_loop(0, ngroups, group, 0)
    plsc.subcore_barrier()

    for k in range(K):
        pltpu.sync_copy(acc_s.at[pl.ds(r0 + k * LANE, LANE)], bufs[0])
        pltpu.sync_copy(bufs[0], acc_out.at[cid, pl.ds(r0 + k * LANE, LANE)])


_sc_agg = pl.kernel(
    _sc_agg_body,
    out_type=_SC_OUT,
    mesh=_MESH,
    scratch_types=[
        pltpu.VMEM_SHARED((N_PAD, D), jnp.float32),
        pltpu.VMEM((GR, LANE), jnp.int32),
        pltpu.VMEM((GR, LANE), jnp.int32),
    ]
    + [pltpu.VMEM((LANE, D), jnp.float32) for _ in range(NB)]
    + [pltpu.SemaphoreType.DMA for _ in range(NB)],
)


def _sc_deg_body(dst_hbm, zeros_hbm, ones_hbm, acc_out,
                 acc_s, dst_v, rows_v):
    cid = lax.axis_index("c")
    sid = lax.axis_index("s")
    wid = cid * NS + sid
    r0 = sid * STRIPE

    pltpu.sync_copy(zeros_hbm, rows_v)
    for k in range(K):
        pltpu.sync_copy(rows_v, acc_s.at[pl.ds(r0 + k * LANE, LANE)])
    plsc.subcore_barrier()

    pltpu.sync_copy(ones_hbm, rows_v)

    def group(g, carry):
        pltpu.sync_copy(dst_hbm.at[wid, pl.ds(g * GR, GR)], dst_v)
        for j in range(GR):
            pltpu.sync_copy(rows_v, acc_s.at[dst_v.at[j]], add=True)
        return carry

    lax.fori_loop(0, CH // GR, group, 0)
    plsc.subcore_barrier()

    for k in range(K):
        pltpu.sync_copy(acc_s.at[pl.ds(r0 + k * LANE, LANE)], rows_v)
        pltpu.sync_copy(rows_v, acc_out.at[cid, pl.ds(r0 + k * LANE, LANE)])


_sc_deg = pl.kernel(
    _sc_deg_body,
    out_type=_SC_OUT,
    mesh=_MESH,
    scratch_types=[
        pltpu.VMEM_SHARED((N_PAD, D), jnp.float32),
        pltpu.VMEM((GR, LANE), jnp.int32),
        pltpu.VMEM((LANE, D), jnp.float32),
    ],
)


def kernel(x, edge_index, W1l, b1l, W1r, W2l, b2l, W2r):
    src = edge_index[0].astype(jnp.int32)
    dst = edge_index[1].astype(jnp.int32)
    pad = E_PAD - N_EDGES
    src_f = jnp.concatenate([src, jnp.zeros((pad,), jnp.int32)])
    dst_f = jnp.concatenate([dst, jnp.full((pad,), N_NODES, jnp.int32)])
    dst_p = dst_f.reshape(NW, CH, LANE)

    CHMX = max(CH_A, CH_B)

    def _split(arr, padval):
        cut = NS * CH_A * LANE
        a = arr[:cut].reshape(NS, CH_A, LANE)
        b = arr[cut:].reshape(NS, CH_B, LANE)
        if CH_A < CHMX:
            a = jnp.concatenate(
                [a, jnp.full((NS, CHMX - CH_A, LANE), padval, jnp.int32)], 1)
        if CH_B < CHMX:
            b = jnp.concatenate(
                [b, jnp.full((NS, CHMX - CH_B, LANE), padval, jnp.int32)], 1)
        return jnp.concatenate([a, b], axis=0)

    src_p = _split(src_f, 0)
    dst_p2 = _split(dst_f, N_NODES)

    xp = jnp.pad(x, ((0, N_PAD - N_NODES), (0, 0)))
    zeros = jnp.zeros((LANE, D), jnp.float32)
    ones = jnp.ones((LANE, D), jnp.float32)
    b1 = b1l.reshape(1, D)
    b2 = b2l.reshape(1, D)

    dega = _sc_deg(dst_p, zeros, ones)

    y1, z1 = _tc_mm(xp, W1l, W1r, b1)
    acc1 = _sc_agg(y1, src_p, dst_p2, zeros)
    y2, z2 = _tc_comb_mm(acc1, dega, z1, W2l, W2r, b2)
    acc2 = _sc_agg(y2, src_p, dst_p2, zeros)
    h2 = _tc_final(acc2, dega, z2)
    return h2[:N_NODES]
